# Initial kernel scaffold; baseline (speedup 1.0000x reference)
import jax
import jax.numpy as jnp
from jax.experimental import pallas as pl


def kernel(x, edge_index, W_f, b_f, W_g, W_ih, W_hh, b_rnn):
    raise NotImplementedError("write your pallas kernel here")


# speedup vs baseline: 2.5391x; 2.5391x over previous
import functools

import jax
import jax.numpy as jnp
from jax import lax
from jax.experimental import pallas as pl
from jax.experimental.pallas import tpu as pltpu
from jax.experimental.pallas import tpu_sc as plsc

N = 10000
E = 320000
D = 128
H = 128

NC = 2
NS = 16
L = 16

R = N // NS
EC = E // NC
C = 4000
NCHUNK = EC // C
K = 128

NEG_INF = float("-inf")


def _tc1_body(x_ref, wf_ref, bf_ref, wih_ref, brnn_ref, wg_ref,
              g_ref, fi_ref):
    feat = jnp.tanh(
        jnp.dot(x_ref[...], wf_ref[...], preferred_element_type=jnp.float32)
        + bf_ref[...])
    fi = jnp.dot(feat, wih_ref[...], preferred_element_type=jnp.float32)
    h0 = jnp.tanh(fi + brnn_ref[...])
    g_ref[...] = jnp.dot(h0, wg_ref[...],
                         preferred_element_type=jnp.float32)
    fi_ref[...] = fi


def _tc1(x, W_f, b_f, W_ih, b_rnn, W_g):
    B = 1000
    grid = (N // B,)
    row_spec = pl.BlockSpec((B, H), lambda i: (i, 0))
    w_spec = pl.BlockSpec((H, H), lambda i: (0, 0))
    v_spec = pl.BlockSpec((1, H), lambda i: (0, 0))
    return pl.pallas_call(
        _tc1_body,
        grid=grid,
        in_specs=[pl.BlockSpec((B, D), lambda i: (i, 0)), w_spec, v_spec,
                  w_spec, v_spec, w_spec],
        out_specs=[row_spec, row_spec],
        out_shape=[jax.ShapeDtypeStruct((N, H), jnp.float32),
                   jax.ShapeDtypeStruct((N, H), jnp.float32)],
    )(x, W_f, b_f.reshape(1, H), W_ih, b_rnn.reshape(1, H), W_g)


def _sc_body(g_hbm, src_hbm, dst_hbm, out_hbm,
             acc, srcb, dstb, cidx, coff, rows, sem1, sem2):
    cid = lax.axis_index("c")
    sid = lax.axis_index("s")
    lo = sid * R

    neg = jnp.full((L,), NEG_INF, dtype=jnp.float32)

    def init_body(i, _):
        acc[pl.ds(i * L, L)] = neg
        return 0

    lax.fori_loop(0, (R + 1) * H // L, init_body, 0)

    def drain_block(d):
        d = pl.multiple_of(d, K)
        pltpu.async_copy(g_hbm.at[cidx.at[pl.ds(d, K)]], rows, sem1).wait()

        def grp_body(jj, _):
            offv = coff[pl.ds(d + jj * L, L)]
            for t in range(L):
                off = pl.multiple_of(offv[t], 8)
                r = jj * L + t
                for c in range(H // L):
                    sl = pl.ds(off + c * L, L)
                    acc[sl] = jnp.maximum(acc[sl], rows[r, pl.ds(c * L, L)])
            return 0

        lax.fori_loop(0, K // L, grp_body, 0)

    def chunk_body(ch, cnt):
        base = pl.multiple_of(cid * EC + ch * C, 8)
        cp1 = pltpu.make_async_copy(src_hbm.at[pl.ds(base, C)], srcb, sem1)
        cp2 = pltpu.make_async_copy(dst_hbm.at[pl.ds(base, C)], dstb, sem2)
        cp1.start()
        cp2.start()
        cp1.wait()
        cp2.wait()

        def filt_body(i, cnt):
            dv = dstb[pl.ds(i * L, L)]
            sv = srcb[pl.ds(i * L, L)]
            m = (dv >= lo) & (dv < lo + R)
            mi = jnp.where(m, jnp.int32(1), jnp.int32(0))
            pref = plsc.cumsum(mi)
            pos = cnt + pref - 1
            plsc.store_scatter(cidx, [pos], sv, mask=m)
            plsc.store_scatter(coff, [pos], (dv - lo) * H, mask=m)
            return cnt + jnp.max(pref)

        cnt = lax.fori_loop(0, C // L, filt_body, cnt)

        def drain_cond(carry):
            d, cnt = carry
            return cnt - d >= K

        def drain_body(carry):
            d, cnt = carry
            drain_block(d)
            return d + K, cnt

        d, cnt = lax.while_loop(drain_cond, drain_body, (jnp.int32(0), cnt))

        d = pl.multiple_of(d, K)
        for t in range(K // L):
            v = cidx[pl.ds(d + t * L, L)]
            cidx[pl.ds(t * L, L)] = v
            w = coff[pl.ds(d + t * L, L)]
            coff[pl.ds(t * L, L)] = w
        return cnt - d

    cnt = lax.fori_loop(0, NCHUNK, chunk_body, jnp.int32(0))

    lane = lax.broadcasted_iota(jnp.int32, (L,), 0)
    pad_idx = lane + sid * L
    pad_off = jnp.full((L,), R * H, dtype=jnp.int32)
    for t in range(K // L):
        pos = cnt + t * L + lane
        plsc.store_scatter(cidx, [pos], pad_idx)
        plsc.store_scatter(coff, [pos], pad_off)
    drain_block(jnp.int32(0))

    pltpu.sync_copy(acc.at[pl.ds(0, R * H)],
                    out_hbm.at[cid, pl.ds(pl.multiple_of(lo * H, 8), R * H)])


def _sc_segmax(g, src, dst):
    mesh = plsc.VectorSubcoreMesh(core_axis_name="c", subcore_axis_name="s")
    kern = pl.kernel(
        _sc_body,
        out_type=jax.ShapeDtypeStruct((NC, N * H), jnp.float32),
        mesh=mesh,
        compiler_params=pltpu.CompilerParams(needs_layout_passes=False),
        scratch_types=[
            pltpu.VMEM(((R + 1) * H,), jnp.float32),
            pltpu.VMEM((C,), jnp.int32),
            pltpu.VMEM((C,), jnp.int32),
            pltpu.VMEM((C + K + L,), jnp.int32),
            pltpu.VMEM((C + K + L,), jnp.int32),
            pltpu.VMEM((K, H), jnp.float32),
            pltpu.SemaphoreType.DMA,
            pltpu.SemaphoreType.DMA,
        ],
    )
    return kern(g, src, dst)


def _tc2_body(fi_ref, a0_ref, a1_ref, whh_ref, brnn_ref, h_ref):
    m = jnp.maximum(a0_ref[...], a1_ref[...])
    m = jnp.where(m == NEG_INF, 0.0, m)
    h_ref[...] = jnp.tanh(
        fi_ref[...]
        + jnp.dot(m, whh_ref[...], preferred_element_type=jnp.float32)
        + brnn_ref[...])


def _tc2(fi, a0, a1, W_hh, b_rnn):
    B = 1000
    grid = (N // B,)
    row_spec = pl.BlockSpec((B, H), lambda i: (i, 0))
    w_spec = pl.BlockSpec((H, H), lambda i: (0, 0))
    v_spec = pl.BlockSpec((1, H), lambda i: (0, 0))
    return pl.pallas_call(
        _tc2_body,
        grid=grid,
        in_specs=[row_spec, row_spec, row_spec, w_spec, v_spec],
        out_specs=row_spec,
        out_shape=jax.ShapeDtypeStruct((N, H), jnp.float32),
    )(fi, a0, a1, W_hh, b_rnn.reshape(1, H))


def kernel(x, edge_index, W_f, b_f, W_g, W_ih, W_hh, b_rnn):
    g, fi = _tc1(x, W_f, b_f, W_ih, b_rnn, W_g)
    src = edge_index[0]
    dst = edge_index[1]
    agg2 = _sc_segmax(g, src, dst)
    a0 = agg2[0].reshape(N, H)
    a1 = agg2[1].reshape(N, H)
    return _tc2(fi, a0, a1, W_hh, b_rnn)
